# Initial kernel scaffold; baseline (speedup 1.0000x reference)
import jax
import jax.numpy as jnp
from jax.experimental import pallas as pl


def kernel(x_all, edge_index_all, batch_index, W1, b1, g1, bt1, W2, b2, g2, bt2, W3, b3, g3, bt3, Wfc, bfc):
    raise NotImplementedError("write your pallas kernel here")


# speedup vs baseline: 6.6764x; 6.6764x over previous
import functools

import jax
import jax.numpy as jnp
from jax import lax
from jax.experimental import pallas as pl
from jax.experimental.pallas import tpu as pltpu
from jax.experimental.pallas import tpu_sc as plsc

N = 10000
E = 320000
D = 128
H = 128
OUT = 2560
B = 64
EPS = 1e-5

NP = 10240
NC = 2
NS = 16
NW = NC * NS
CH = 80
NCH = 128
EPAD = NW * NCH * CH
RPT = NP // NS
BM = 1280
NB = NP // BM
SEG = 128
CHP = 40
NCHP = NP // NW // CHP

def _zero16():
    return jnp.zeros((16,), jnp.float32)


def _deg_body(dst_hbm, out_hbm, dstv, onesv, zerov, dacc):
    cid = lax.axis_index("c")
    sid = lax.axis_index("s")
    wid = sid * NC + cid

    def fill_ones(i, carry):
        onesv[pl.ds(i * 16, 16)] = jnp.full((16,), 1.0, jnp.float32)
        return carry
    lax.fori_loop(0, CH // 16, fill_ones, 0)

    def fill_zero(i, carry):
        zerov[pl.ds(i * 16, 16)] = _zero16()
        return carry
    lax.fori_loop(0, RPT // 16, fill_zero, 0)

    pltpu.sync_copy(zerov, dacc.at[pl.ds(sid * RPT, RPT)])
    pltpu.sync_copy(dst_hbm.at[pl.ds(wid * NCH, NCH)], dstv)
    plsc.subcore_barrier()

    def body(g, carry):
        pltpu.sync_copy(onesv, dacc.at[dstv.at[g]], add=True)
        return carry
    lax.fori_loop(0, NCH, body, 0)

    plsc.subcore_barrier()
    pltpu.sync_copy(dacc.at[pl.ds(sid * RPT, RPT)],
                    out_hbm.at[pl.ds(cid * NP + sid * RPT, RPT)])


@functools.cache
def _deg():
    return functools.partial(
        pl.kernel,
        mesh=plsc.VectorSubcoreMesh(core_axis_name="c", subcore_axis_name="s"),
        out_type=jax.ShapeDtypeStruct((NC * NP,), jnp.float32),
        scratch_types=[
            pltpu.VMEM((NCH, CH), jnp.int32),
            pltpu.VMEM((CH,), jnp.float32),
            pltpu.VMEM((RPT,), jnp.float32),
            pltpu.VMEM_SHARED((NP,), jnp.float32),
        ],
    )(_deg_body)


def _agg_body(y_hbm, src_hbm, dst_hbm, out_hbm, srcv, dstv, rowsv, acc, sem):
    cid = lax.axis_index("c")
    sid = lax.axis_index("s")
    wid = sid * NC + cid

    def fill_zero(i, carry):
        rowsv[i // 8, pl.ds((i % 8) * 16, 16)] = _zero16()
        return carry
    lax.fori_loop(0, CH * 8, fill_zero, 0)

    def zero_acc(k, carry):
        pltpu.sync_copy(rowsv, acc.at[pl.ds(sid * RPT + k * CH, CH)])
        return carry
    lax.fori_loop(0, RPT // CH, zero_acc, 0)

    pltpu.sync_copy(src_hbm.at[pl.ds(wid * NCH, NCH)], srcv)
    pltpu.sync_copy(dst_hbm.at[pl.ds(wid * NCH, NCH)], dstv)
    plsc.subcore_barrier()

    def body(g, carry):
        pltpu.async_copy(y_hbm.at[srcv.at[g]], rowsv, sem).wait()
        pltpu.sync_copy(rowsv, acc.at[dstv.at[g]], add=True)
        return carry
    lax.fori_loop(0, NCH, body, 0)

    plsc.subcore_barrier()
    pltpu.sync_copy(acc.at[pl.ds(sid * RPT, RPT)],
                    out_hbm.at[pl.ds(cid * NP + sid * RPT, RPT)])


@functools.cache
def _agg():
    return functools.partial(
        pl.kernel,
        mesh=plsc.VectorSubcoreMesh(core_axis_name="c", subcore_axis_name="s"),
        out_type=jax.ShapeDtypeStruct((NC * NP, D), jnp.float32),
        scratch_types=[
            pltpu.VMEM((NCH, CH), jnp.int32),
            pltpu.VMEM((NCH, CH), jnp.int32),
            pltpu.VMEM((CH, D), jnp.float32),
            pltpu.VMEM_SHARED((NP, D), jnp.float32),
            pltpu.SemaphoreType.DMA,
        ],
    )(_agg_body)


def _pool_body(h_hbm, bidx_hbm, out_hbm, bidxv, hrows, onesr, pacc, cacc):
    cid = lax.axis_index("c")
    sid = lax.axis_index("s")
    wid = sid * NC + cid
    spt = SEG // NS

    def fill_zero(i, carry):
        hrows[i // 8, pl.ds((i % 8) * 16, 16)] = _zero16()
        return carry
    lax.fori_loop(0, CHP * 8, fill_zero, 0)
    pltpu.sync_copy(hrows.at[pl.ds(0, spt)], pacc.at[pl.ds(sid * spt, spt)])
    pltpu.sync_copy(hrows.at[pl.ds(0, spt)], cacc.at[pl.ds(sid * spt, spt)])

    def fill_ones(i, carry):
        onesr[i // 8, pl.ds((i % 8) * 16, 16)] = jnp.full((16,), 1.0, jnp.float32)
        return carry
    lax.fori_loop(0, CHP * 8, fill_ones, 0)

    pltpu.sync_copy(bidx_hbm.at[pl.ds(wid * NCHP, NCHP)], bidxv)
    plsc.subcore_barrier()

    def body(g, carry):
        pltpu.sync_copy(
            h_hbm.at[pl.ds(wid * (NP // NW) + g * CHP, CHP)], hrows)
        pltpu.sync_copy(hrows, pacc.at[bidxv.at[g]], add=True)
        pltpu.sync_copy(onesr, cacc.at[bidxv.at[g]], add=True)
        return carry
    lax.fori_loop(0, NCHP, body, 0)

    plsc.subcore_barrier()
    pltpu.sync_copy(pacc.at[pl.ds(sid * spt, spt)],
                    out_hbm.at[pl.ds(cid * 2 * SEG + sid * spt, spt)])
    pltpu.sync_copy(cacc.at[pl.ds(sid * spt, spt)],
                    out_hbm.at[pl.ds(cid * 2 * SEG + SEG + sid * spt, spt)])


@functools.cache
def _pool():
    return functools.partial(
        pl.kernel,
        mesh=plsc.VectorSubcoreMesh(core_axis_name="c", subcore_axis_name="s"),
        out_type=jax.ShapeDtypeStruct((NC * 2 * SEG, D), jnp.float32),
        scratch_types=[
            pltpu.VMEM((NCHP, CHP), jnp.int32),
            pltpu.VMEM((CHP, D), jnp.float32),
            pltpu.VMEM((CHP, D), jnp.float32),
            pltpu.VMEM_SHARED((SEG, D), jnp.float32),
            pltpu.VMEM_SHARED((SEG, D), jnp.float32),
        ],
    )(_pool_body)


def _mm_scaled_body(x_ref, w_ref, deg_ref, y_ref):
    deg = jnp.sum(deg_ref[...], axis=0) + 1.0
    dinv = lax.rsqrt(deg)
    y_ref[...] = jnp.dot(x_ref[...], w_ref[...],
                         preferred_element_type=jnp.float32) * dinv[:, None]


def _mm_scaled(x, w, deg2):
    return pl.pallas_call(
        _mm_scaled_body,
        grid=(NB,),
        in_specs=[
            pl.BlockSpec((BM, D), lambda i: (i, 0)),
            pl.BlockSpec((D, H), lambda i: (0, 0)),
            pl.BlockSpec((NC, BM), lambda i: (0, i)),
        ],
        out_specs=pl.BlockSpec((BM, H), lambda i: (i, 0)),
        out_shape=jax.ShapeDtypeStruct((NP, H), jnp.float32),
    )(x, w, deg2)


def _stats_body(accs_ref, y_ref, deg_ref, b_ref, z_ref, ps_ref, pq_ref):
    i = pl.program_id(0)
    deg = jnp.sum(deg_ref[...], axis=0) + 1.0
    dinv = lax.rsqrt(deg)
    a = accs_ref[0] + accs_ref[1] + y_ref[...]
    z = a * dinv[:, None] + b_ref[...]
    z_ref[...] = z
    ridx = lax.broadcasted_iota(jnp.int32, (BM, 1), 0) + i * BM
    zm = jnp.where(ridx < N, z, 0.0)
    ps_ref[...] = jnp.sum(zm, axis=0).reshape(1, 1, H)
    pq_ref[...] = jnp.sum(zm * zm, axis=0).reshape(1, 1, H)


def _stats(accs, y, deg2, b):
    return pl.pallas_call(
        _stats_body,
        grid=(NB,),
        in_specs=[
            pl.BlockSpec((NC, BM, H), lambda i: (0, i, 0)),
            pl.BlockSpec((BM, H), lambda i: (i, 0)),
            pl.BlockSpec((NC, BM), lambda i: (0, i)),
            pl.BlockSpec((1, H), lambda i: (0, 0)),
        ],
        out_specs=[
            pl.BlockSpec((BM, H), lambda i: (i, 0)),
            pl.BlockSpec((1, 1, H), lambda i: (i, 0, 0)),
            pl.BlockSpec((1, 1, H), lambda i: (i, 0, 0)),
        ],
        out_shape=[
            jax.ShapeDtypeStruct((NP, H), jnp.float32),
            jax.ShapeDtypeStruct((NB, 1, H), jnp.float32),
            jax.ShapeDtypeStruct((NB, 1, H), jnp.float32),
        ],
    )(accs, y, deg2, b)


def _bn_common(z_ref, ps_ref, pq_ref, g_ref, bt_ref):
    s = jnp.sum(ps_ref[...], axis=(0, 1))
    q = jnp.sum(pq_ref[...], axis=(0, 1))
    mean = s * (1.0 / N)
    var = q * (1.0 / N) - mean * mean
    scale = lax.rsqrt(var + EPS) * g_ref[0]
    return jnp.maximum((z_ref[...] - mean) * scale + bt_ref[0], 0.0)


def _norm_mm_body(z_ref, ps_ref, pq_ref, g_ref, bt_ref, w_ref, deg_ref, y_ref):
    h = _bn_common(z_ref, ps_ref, pq_ref, g_ref, bt_ref)
    deg = jnp.sum(deg_ref[...], axis=0) + 1.0
    dinv = lax.rsqrt(deg)
    y_ref[...] = jnp.dot(h, w_ref[...],
                         preferred_element_type=jnp.float32) * dinv[:, None]


def _norm_mm(z, ps, pq, g, bt, w, deg2):
    return pl.pallas_call(
        _norm_mm_body,
        grid=(NB,),
        in_specs=[
            pl.BlockSpec((BM, H), lambda i: (i, 0)),
            pl.BlockSpec((NB, 1, H), lambda i: (0, 0, 0)),
            pl.BlockSpec((NB, 1, H), lambda i: (0, 0, 0)),
            pl.BlockSpec((1, H), lambda i: (0, 0)),
            pl.BlockSpec((1, H), lambda i: (0, 0)),
            pl.BlockSpec((H, H), lambda i: (0, 0)),
            pl.BlockSpec((NC, BM), lambda i: (0, i)),
        ],
        out_specs=pl.BlockSpec((BM, H), lambda i: (i, 0)),
        out_shape=jax.ShapeDtypeStruct((NP, H), jnp.float32),
    )(z, ps, pq, g, bt, w, deg2)


def _norm_only_body(z_ref, ps_ref, pq_ref, g_ref, bt_ref, h_ref):
    h_ref[...] = _bn_common(z_ref, ps_ref, pq_ref, g_ref, bt_ref)


def _norm_only(z, ps, pq, g, bt):
    return pl.pallas_call(
        _norm_only_body,
        grid=(NB,),
        in_specs=[
            pl.BlockSpec((BM, H), lambda i: (i, 0)),
            pl.BlockSpec((NB, 1, H), lambda i: (0, 0, 0)),
            pl.BlockSpec((NB, 1, H), lambda i: (0, 0, 0)),
            pl.BlockSpec((1, H), lambda i: (0, 0)),
            pl.BlockSpec((1, H), lambda i: (0, 0)),
        ],
        out_specs=pl.BlockSpec((BM, H), lambda i: (i, 0)),
        out_shape=jax.ShapeDtypeStruct((NP, H), jnp.float32),
    )(z, ps, pq, g, bt)


def _fc_body(p0_ref, c0_ref, p1_ref, c1_ref, w_ref, b_ref, o_ref):
    psum = p0_ref[...] + p1_ref[...]
    cnt = jnp.maximum(c0_ref[...] + c1_ref[...], 1.0)
    pool = psum / cnt
    o_ref[...] = jnp.maximum(
        jnp.dot(pool[:B], w_ref[...], preferred_element_type=jnp.float32)
        + b_ref[...], 0.0)


def _fc(p0, c0, p1, c1, w, b):
    return pl.pallas_call(
        _fc_body,
        grid=(1,),
        in_specs=[
            pl.BlockSpec((SEG, H), lambda i: (0, 0)),
            pl.BlockSpec((SEG, H), lambda i: (0, 0)),
            pl.BlockSpec((SEG, H), lambda i: (0, 0)),
            pl.BlockSpec((SEG, H), lambda i: (0, 0)),
            pl.BlockSpec((H, OUT), lambda i: (0, 0)),
            pl.BlockSpec((1, OUT), lambda i: (0, 0)),
        ],
        out_specs=pl.BlockSpec((B, OUT), lambda i: (0, 0)),
        out_shape=jax.ShapeDtypeStruct((B, OUT), jnp.float32),
    )(p0, c0, p1, c1, w, b)


def kernel(x_all, edge_index_all, batch_index, W1, b1, g1, bt1, W2, b2, g2,
           bt2, W3, b3, g3, bt3, Wfc, bfc):
    epad = jnp.full((2, EPAD - E), N, jnp.int32)
    ei = jnp.concatenate([edge_index_all, epad], axis=1)
    src2 = ei[0].reshape(EPAD // CH, CH)
    dst2 = ei[1].reshape(EPAD // CH, CH)
    x_pad = jnp.concatenate(
        [x_all.astype(jnp.float32), jnp.zeros((NP - N, D), jnp.float32)], 0)
    bidx2 = jnp.concatenate(
        [batch_index, jnp.full((NP - N,), B, jnp.int32)], 0).reshape(
            NP // CHP, CHP)

    deg2 = _deg()(dst2).reshape(NC, NP)

    b1r, g1r, bt1r = b1.reshape(1, H), g1.reshape(1, H), bt1.reshape(1, H)
    b2r, g2r, bt2r = b2.reshape(1, H), g2.reshape(1, H), bt2.reshape(1, H)
    b3r, g3r, bt3r = b3.reshape(1, H), g3.reshape(1, H), bt3.reshape(1, H)

    y1 = _mm_scaled(x_pad, W1, deg2)
    acc1 = _agg()(y1, src2, dst2).reshape(NC, NP, D)
    z1, ps1, pq1 = _stats(acc1, y1, deg2, b1r)
    y2 = _norm_mm(z1, ps1, pq1, g1r, bt1r, W2, deg2)

    acc2 = _agg()(y2, src2, dst2).reshape(NC, NP, D)
    z2, ps2, pq2 = _stats(acc2, y2, deg2, b2r)
    y3 = _norm_mm(z2, ps2, pq2, g2r, bt2r, W3, deg2)

    acc3 = _agg()(y3, src2, dst2).reshape(NC, NP, D)
    z3, ps3, pq3 = _stats(acc3, y3, deg2, b3r)
    h3 = _norm_only(z3, ps3, pq3, g3r, bt3r)

    pool_flat = _pool()(h3, bidx2)
    p0 = pool_flat[0:SEG]
    c0 = pool_flat[SEG:2 * SEG]
    p1 = pool_flat[2 * SEG:3 * SEG]
    c1 = pool_flat[3 * SEG:4 * SEG]

    return _fc(p0, c0, p1, c1, Wfc, bfc.reshape(1, OUT))


# speedup vs baseline: 7.7197x; 1.1563x over previous
import functools

import jax
import jax.numpy as jnp
from jax import lax
from jax.experimental import pallas as pl
from jax.experimental.pallas import tpu as pltpu
from jax.experimental.pallas import tpu_sc as plsc

N = 10000
E = 320000
D = 128
H = 128
OUT = 2560
B = 64
EPS = 1e-5

NP = 10240
NC = 2
NS = 16
NW = NC * NS
CH = 80
NCH = 128
QCH = 32
EPAD = NW * NCH * CH
RPT = NP // NS
BM = 1280
NB = NP // BM
SEG = 128
CHP = 40
NCHP = NP // NW // CHP

def _zero16():
    return jnp.zeros((16,), jnp.float32)


def _deg_body(dst_hbm, out_hbm, dstv, onesv, zerov, dacc):
    cid = lax.axis_index("c")
    sid = lax.axis_index("s")
    wid = sid * NC + cid

    def fill_ones(i, carry):
        onesv[pl.ds(i * 16, 16)] = jnp.full((16,), 1.0, jnp.float32)
        return carry
    lax.fori_loop(0, CH // 16, fill_ones, 0)

    def fill_zero(i, carry):
        zerov[pl.ds(i * 16, 16)] = _zero16()
        return carry
    lax.fori_loop(0, RPT // 16, fill_zero, 0)

    pltpu.sync_copy(zerov, dacc.at[pl.ds(sid * RPT, RPT)])
    pltpu.sync_copy(dst_hbm.at[pl.ds(wid * NCH, NCH)], dstv)
    plsc.subcore_barrier()

    def body(g, carry):
        pltpu.sync_copy(onesv, dacc.at[dstv.at[g]], add=True)
        return carry
    lax.fori_loop(0, NCH, body, 0)

    plsc.subcore_barrier()
    pltpu.sync_copy(dacc.at[pl.ds(sid * RPT, RPT)],
                    out_hbm.at[pl.ds(cid * NP + sid * RPT, RPT)])


@functools.cache
def _deg():
    return functools.partial(
        pl.kernel,
        mesh=plsc.VectorSubcoreMesh(core_axis_name="c", subcore_axis_name="s"),
        out_type=jax.ShapeDtypeStruct((NC * NP,), jnp.float32),
        scratch_types=[
            pltpu.VMEM((NCH, CH), jnp.int32),
            pltpu.VMEM((CH,), jnp.float32),
            pltpu.VMEM((RPT,), jnp.float32),
            pltpu.VMEM_SHARED((NP,), jnp.float32),
        ],
    )(_deg_body)


def _agg_body(y_hbm, src_hbm, dst_hbm, out_hbm, srcv, dstv, rows0, rows1,
              acc, sem0, sem1):
    cid = lax.axis_index("c")
    sid = lax.axis_index("s")
    wid = sid * NC + cid

    def fill_zero(i, carry):
        rows0[i // 8, pl.ds((i % 8) * 16, 16)] = _zero16()
        return carry
    lax.fori_loop(0, CH * 8, fill_zero, 0)

    def zero_acc(k, carry):
        pltpu.sync_copy(rows0, acc.at[pl.ds(sid * RPT + k * CH, CH)])
        return carry
    lax.fori_loop(0, RPT // CH, zero_acc, 0)

    plsc.subcore_barrier()

    def seg(s, carry):
        pltpu.sync_copy(src_hbm.at[pl.ds(wid * NCH + s * QCH, QCH)], srcv)
        pltpu.sync_copy(dst_hbm.at[pl.ds(wid * NCH + s * QCH, QCH)], dstv)
        pltpu.make_async_copy(y_hbm.at[srcv.at[0]], rows0, sem0).start()

        def body(t, c2):
            g0 = 2 * t
            g1 = g0 + 1
            pltpu.make_async_copy(y_hbm.at[srcv.at[g1]], rows1, sem1).start()
            pltpu.make_async_copy(y_hbm.at[srcv.at[g0]], rows0, sem0).wait()
            pltpu.sync_copy(rows0, acc.at[dstv.at[g0]], add=True)

            @pl.when(g1 + 1 < QCH)
            def _():
                pltpu.make_async_copy(y_hbm.at[srcv.at[g1 + 1]], rows0,
                                      sem0).start()
            pltpu.make_async_copy(y_hbm.at[srcv.at[g1]], rows1, sem1).wait()
            pltpu.sync_copy(rows1, acc.at[dstv.at[g1]], add=True)
            return c2
        lax.fori_loop(0, QCH // 2, body, 0)
        return carry
    lax.fori_loop(0, NCH // QCH, seg, 0)

    plsc.subcore_barrier()
    pltpu.sync_copy(acc.at[pl.ds(sid * RPT, RPT)],
                    out_hbm.at[pl.ds(cid * NP + sid * RPT, RPT)])


@functools.cache
def _agg():
    return functools.partial(
        pl.kernel,
        mesh=plsc.VectorSubcoreMesh(core_axis_name="c", subcore_axis_name="s"),
        out_type=jax.ShapeDtypeStruct((NC * NP, D), jnp.float32),
        scratch_types=[
            pltpu.VMEM((QCH, CH), jnp.int32),
            pltpu.VMEM((QCH, CH), jnp.int32),
            pltpu.VMEM((CH, D), jnp.float32),
            pltpu.VMEM((CH, D), jnp.float32),
            pltpu.VMEM_SHARED((NP, D), jnp.float32),
            pltpu.SemaphoreType.DMA,
            pltpu.SemaphoreType.DMA,
        ],
    )(_agg_body)


def _pool_body(h_hbm, bidx_hbm, out_hbm, bidxv, hrows, onesr, pacc, cacc):
    cid = lax.axis_index("c")
    sid = lax.axis_index("s")
    wid = sid * NC + cid
    spt = SEG // NS

    def fill_zero(i, carry):
        hrows[i // 8, pl.ds((i % 8) * 16, 16)] = _zero16()
        return carry
    lax.fori_loop(0, CHP * 8, fill_zero, 0)
    pltpu.sync_copy(hrows.at[pl.ds(0, spt)], pacc.at[pl.ds(sid * spt, spt)])
    pltpu.sync_copy(hrows.at[pl.ds(0, spt)], cacc.at[pl.ds(sid * spt, spt)])

    def fill_ones(i, carry):
        onesr[i // 8, pl.ds((i % 8) * 16, 16)] = jnp.full((16,), 1.0, jnp.float32)
        return carry
    lax.fori_loop(0, CHP * 8, fill_ones, 0)

    pltpu.sync_copy(bidx_hbm.at[pl.ds(wid * NCHP, NCHP)], bidxv)
    plsc.subcore_barrier()

    def body(g, carry):
        pltpu.sync_copy(
            h_hbm.at[pl.ds(wid * (NP // NW) + g * CHP, CHP)], hrows)
        pltpu.sync_copy(hrows, pacc.at[bidxv.at[g]], add=True)
        pltpu.sync_copy(onesr, cacc.at[bidxv.at[g]], add=True)
        return carry
    lax.fori_loop(0, NCHP, body, 0)

    plsc.subcore_barrier()
    pltpu.sync_copy(pacc.at[pl.ds(sid * spt, spt)],
                    out_hbm.at[pl.ds(cid * 2 * SEG + sid * spt, spt)])
    pltpu.sync_copy(cacc.at[pl.ds(sid * spt, spt)],
                    out_hbm.at[pl.ds(cid * 2 * SEG + SEG + sid * spt, spt)])


@functools.cache
def _pool():
    return functools.partial(
        pl.kernel,
        mesh=plsc.VectorSubcoreMesh(core_axis_name="c", subcore_axis_name="s"),
        out_type=jax.ShapeDtypeStruct((NC * 2 * SEG, D), jnp.float32),
        scratch_types=[
            pltpu.VMEM((NCHP, CHP), jnp.int32),
            pltpu.VMEM((CHP, D), jnp.float32),
            pltpu.VMEM((CHP, D), jnp.float32),
            pltpu.VMEM_SHARED((SEG, D), jnp.float32),
            pltpu.VMEM_SHARED((SEG, D), jnp.float32),
        ],
    )(_pool_body)


def _mm_scaled_body(x_ref, w_ref, deg_ref, y_ref):
    deg = jnp.sum(deg_ref[...], axis=0) + 1.0
    dinv = lax.rsqrt(deg)
    y_ref[...] = jnp.dot(x_ref[...], w_ref[...],
                         preferred_element_type=jnp.float32) * dinv[:, None]


def _mm_scaled(x, w, deg2):
    return pl.pallas_call(
        _mm_scaled_body,
        grid=(NB,),
        in_specs=[
            pl.BlockSpec((BM, D), lambda i: (i, 0)),
            pl.BlockSpec((D, H), lambda i: (0, 0)),
            pl.BlockSpec((NC, BM), lambda i: (0, i)),
        ],
        out_specs=pl.BlockSpec((BM, H), lambda i: (i, 0)),
        out_shape=jax.ShapeDtypeStruct((NP, H), jnp.float32),
    )(x, w, deg2)


def _stats_body(accs_ref, y_ref, deg_ref, b_ref, z_ref, ps_ref, pq_ref):
    i = pl.program_id(0)
    deg = jnp.sum(deg_ref[...], axis=0) + 1.0
    dinv = lax.rsqrt(deg)
    a = accs_ref[0] + accs_ref[1] + y_ref[...]
    z = a * dinv[:, None] + b_ref[...]
    z_ref[...] = z
    ridx = lax.broadcasted_iota(jnp.int32, (BM, 1), 0) + i * BM
    zm = jnp.where(ridx < N, z, 0.0)
    ps_ref[...] = jnp.sum(zm, axis=0).reshape(1, 1, H)
    pq_ref[...] = jnp.sum(zm * zm, axis=0).reshape(1, 1, H)


def _stats(accs, y, deg2, b):
    return pl.pallas_call(
        _stats_body,
        grid=(NB,),
        in_specs=[
            pl.BlockSpec((NC, BM, H), lambda i: (0, i, 0)),
            pl.BlockSpec((BM, H), lambda i: (i, 0)),
            pl.BlockSpec((NC, BM), lambda i: (0, i)),
            pl.BlockSpec((1, H), lambda i: (0, 0)),
        ],
        out_specs=[
            pl.BlockSpec((BM, H), lambda i: (i, 0)),
            pl.BlockSpec((1, 1, H), lambda i: (i, 0, 0)),
            pl.BlockSpec((1, 1, H), lambda i: (i, 0, 0)),
        ],
        out_shape=[
            jax.ShapeDtypeStruct((NP, H), jnp.float32),
            jax.ShapeDtypeStruct((NB, 1, H), jnp.float32),
            jax.ShapeDtypeStruct((NB, 1, H), jnp.float32),
        ],
    )(accs, y, deg2, b)


def _bn_common(z_ref, ps_ref, pq_ref, g_ref, bt_ref):
    s = jnp.sum(ps_ref[...], axis=(0, 1))
    q = jnp.sum(pq_ref[...], axis=(0, 1))
    mean = s * (1.0 / N)
    var = q * (1.0 / N) - mean * mean
    scale = lax.rsqrt(var + EPS) * g_ref[0]
    return jnp.maximum((z_ref[...] - mean) * scale + bt_ref[0], 0.0)


def _norm_mm_body(z_ref, ps_ref, pq_ref, g_ref, bt_ref, w_ref, deg_ref, y_ref):
    h = _bn_common(z_ref, ps_ref, pq_ref, g_ref, bt_ref)
    deg = jnp.sum(deg_ref[...], axis=0) + 1.0
    dinv = lax.rsqrt(deg)
    y_ref[...] = jnp.dot(h, w_ref[...],
                         preferred_element_type=jnp.float32) * dinv[:, None]


def _norm_mm(z, ps, pq, g, bt, w, deg2):
    return pl.pallas_call(
        _norm_mm_body,
        grid=(NB,),
        in_specs=[
            pl.BlockSpec((BM, H), lambda i: (i, 0)),
            pl.BlockSpec((NB, 1, H), lambda i: (0, 0, 0)),
            pl.BlockSpec((NB, 1, H), lambda i: (0, 0, 0)),
            pl.BlockSpec((1, H), lambda i: (0, 0)),
            pl.BlockSpec((1, H), lambda i: (0, 0)),
            pl.BlockSpec((H, H), lambda i: (0, 0)),
            pl.BlockSpec((NC, BM), lambda i: (0, i)),
        ],
        out_specs=pl.BlockSpec((BM, H), lambda i: (i, 0)),
        out_shape=jax.ShapeDtypeStruct((NP, H), jnp.float32),
    )(z, ps, pq, g, bt, w, deg2)


def _norm_only_body(z_ref, ps_ref, pq_ref, g_ref, bt_ref, h_ref):
    h_ref[...] = _bn_common(z_ref, ps_ref, pq_ref, g_ref, bt_ref)


def _norm_only(z, ps, pq, g, bt):
    return pl.pallas_call(
        _norm_only_body,
        grid=(NB,),
        in_specs=[
            pl.BlockSpec((BM, H), lambda i: (i, 0)),
            pl.BlockSpec((NB, 1, H), lambda i: (0, 0, 0)),
            pl.BlockSpec((NB, 1, H), lambda i: (0, 0, 0)),
            pl.BlockSpec((1, H), lambda i: (0, 0)),
            pl.BlockSpec((1, H), lambda i: (0, 0)),
        ],
        out_specs=pl.BlockSpec((BM, H), lambda i: (i, 0)),
        out_shape=jax.ShapeDtypeStruct((NP, H), jnp.float32),
    )(z, ps, pq, g, bt)


def _fc_body(p0_ref, c0_ref, p1_ref, c1_ref, w_ref, b_ref, o_ref):
    psum = p0_ref[...] + p1_ref[...]
    cnt = jnp.maximum(c0_ref[...] + c1_ref[...], 1.0)
    pool = psum / cnt
    o_ref[...] = jnp.maximum(
        jnp.dot(pool[:B], w_ref[...], preferred_element_type=jnp.float32)
        + b_ref[...], 0.0)


def _fc(p0, c0, p1, c1, w, b):
    return pl.pallas_call(
        _fc_body,
        grid=(1,),
        in_specs=[
            pl.BlockSpec((SEG, H), lambda i: (0, 0)),
            pl.BlockSpec((SEG, H), lambda i: (0, 0)),
            pl.BlockSpec((SEG, H), lambda i: (0, 0)),
            pl.BlockSpec((SEG, H), lambda i: (0, 0)),
            pl.BlockSpec((H, OUT), lambda i: (0, 0)),
            pl.BlockSpec((1, OUT), lambda i: (0, 0)),
        ],
        out_specs=pl.BlockSpec((B, OUT), lambda i: (0, 0)),
        out_shape=jax.ShapeDtypeStruct((B, OUT), jnp.float32),
    )(p0, c0, p1, c1, w, b)


def kernel(x_all, edge_index_all, batch_index, W1, b1, g1, bt1, W2, b2, g2,
           bt2, W3, b3, g3, bt3, Wfc, bfc):
    epad = jnp.full((2, EPAD - E), N, jnp.int32)
    ei = jnp.concatenate([edge_index_all, epad], axis=1)
    src2 = ei[0].reshape(EPAD // CH, CH)
    dst2 = ei[1].reshape(EPAD // CH, CH)
    x_pad = jnp.concatenate(
        [x_all.astype(jnp.float32), jnp.zeros((NP - N, D), jnp.float32)], 0)
    bidx2 = jnp.concatenate(
        [batch_index, jnp.full((NP - N,), B, jnp.int32)], 0).reshape(
            NP // CHP, CHP)

    deg2 = _deg()(dst2).reshape(NC, NP)

    b1r, g1r, bt1r = b1.reshape(1, H), g1.reshape(1, H), bt1.reshape(1, H)
    b2r, g2r, bt2r = b2.reshape(1, H), g2.reshape(1, H), bt2.reshape(1, H)
    b3r, g3r, bt3r = b3.reshape(1, H), g3.reshape(1, H), bt3.reshape(1, H)

    y1 = _mm_scaled(x_pad, W1, deg2)
    acc1 = _agg()(y1, src2, dst2).reshape(NC, NP, D)
    z1, ps1, pq1 = _stats(acc1, y1, deg2, b1r)
    y2 = _norm_mm(z1, ps1, pq1, g1r, bt1r, W2, deg2)

    acc2 = _agg()(y2, src2, dst2).reshape(NC, NP, D)
    z2, ps2, pq2 = _stats(acc2, y2, deg2, b2r)
    y3 = _norm_mm(z2, ps2, pq2, g2r, bt2r, W3, deg2)

    acc3 = _agg()(y3, src2, dst2).reshape(NC, NP, D)
    z3, ps3, pq3 = _stats(acc3, y3, deg2, b3r)
    h3 = _norm_only(z3, ps3, pq3, g3r, bt3r)

    pool_flat = _pool()(h3, bidx2)
    p0 = pool_flat[0:SEG]
    c0 = pool_flat[SEG:2 * SEG]
    p1 = pool_flat[2 * SEG:3 * SEG]
    c1 = pool_flat[3 * SEG:4 * SEG]

    return _fc(p0, c0, p1, c1, Wfc, bfc.reshape(1, OUT))


# speedup vs baseline: 8.8917x; 1.1518x over previous
import functools

import jax
import jax.numpy as jnp
from jax import lax
from jax.experimental import pallas as pl
from jax.experimental.pallas import tpu as pltpu
from jax.experimental.pallas import tpu_sc as plsc

N = 10000
E = 320000
D = 128
H = 128
OUT = 2560
B = 64
EPS = 1e-5

NP = 10240
NC = 2
NS = 16
NW = NC * NS
CH = 80
NCH = 128
QCH = 16
NCH0 = 208
NCH1 = 48
SUBC = NCH0 + NCH1
EPAD = NW * NCH * CH
RPT = NP // NS
BM = 1280
NB = NP // BM
SEG = 128
CHP = 40
NCHP = NP // NW // CHP

def _zero16():
    return jnp.zeros((16,), jnp.float32)


def _deg_body(dst_hbm, out_hbm, dstv, onesv, zerov, dacc):
    cid = lax.axis_index("c")
    sid = lax.axis_index("s")
    wid = sid * NC + cid

    def fill_ones(i, carry):
        onesv[pl.ds(i * 16, 16)] = jnp.full((16,), 1.0, jnp.float32)
        return carry
    lax.fori_loop(0, CH // 16, fill_ones, 0)

    def fill_zero(i, carry):
        zerov[pl.ds(i * 16, 16)] = _zero16()
        return carry
    lax.fori_loop(0, RPT // 16, fill_zero, 0)

    pltpu.sync_copy(zerov, dacc.at[pl.ds(sid * RPT, RPT)])
    pltpu.sync_copy(dst_hbm.at[pl.ds(wid * NCH, NCH)], dstv)
    plsc.subcore_barrier()

    def body(g, carry):
        pltpu.sync_copy(onesv, dacc.at[dstv.at[g]], add=True)
        return carry
    lax.fori_loop(0, NCH, body, 0)

    plsc.subcore_barrier()
    pltpu.sync_copy(dacc.at[pl.ds(sid * RPT, RPT)],
                    out_hbm.at[pl.ds(cid * NP + sid * RPT, RPT)])


@functools.cache
def _deg():
    return functools.partial(
        pl.kernel,
        mesh=plsc.VectorSubcoreMesh(core_axis_name="c", subcore_axis_name="s"),
        out_type=jax.ShapeDtypeStruct((NC * NP,), jnp.float32),
        scratch_types=[
            pltpu.VMEM((NCH, CH), jnp.int32),
            pltpu.VMEM((CH,), jnp.float32),
            pltpu.VMEM((RPT,), jnp.float32),
            pltpu.VMEM_SHARED((NP,), jnp.float32),
        ],
    )(_deg_body)


def _agg_body(y_hbm, src_hbm, dst_hbm, out_hbm, srcv, dstv, rows0, rows1,
              acc, sem0, sem1):
    cid = lax.axis_index("c")
    sid = lax.axis_index("s")
    wid = sid * NC + cid

    def fill_zero(i, carry):
        rows0[i // 8, pl.ds((i % 8) * 16, 16)] = _zero16()
        return carry
    lax.fori_loop(0, CH * 8, fill_zero, 0)

    def zero_acc(k, carry):
        pltpu.sync_copy(rows0, acc.at[pl.ds(sid * RPT + k * CH, CH)])
        return carry
    lax.fori_loop(0, RPT // CH, zero_acc, 0)

    plsc.subcore_barrier()

    nseg = jnp.where(cid == 0, NCH0 // QCH, NCH1 // QCH)
    base = sid * SUBC + cid * NCH0

    def seg(s, carry):
        pltpu.sync_copy(src_hbm.at[pl.ds(base + s * QCH, QCH)], srcv)
        pltpu.sync_copy(dst_hbm.at[pl.ds(base + s * QCH, QCH)], dstv)
        pltpu.make_async_copy(y_hbm.at[srcv.at[0]], rows0, sem0).start()

        def body(t, c2):
            g0 = 2 * t
            g1 = g0 + 1
            pltpu.make_async_copy(y_hbm.at[srcv.at[g1]], rows1, sem1).start()
            pltpu.make_async_copy(y_hbm.at[srcv.at[g0]], rows0, sem0).wait()
            pltpu.sync_copy(rows0, acc.at[dstv.at[g0]], add=True)

            @pl.when(g1 + 1 < QCH)
            def _():
                pltpu.make_async_copy(y_hbm.at[srcv.at[g1 + 1]], rows0,
                                      sem0).start()
            pltpu.make_async_copy(y_hbm.at[srcv.at[g1]], rows1, sem1).wait()
            pltpu.sync_copy(rows1, acc.at[dstv.at[g1]], add=True)
            return c2
        lax.fori_loop(0, QCH // 2, body, 0)
        return carry
    lax.fori_loop(0, nseg, seg, 0)

    plsc.subcore_barrier()
    pltpu.sync_copy(acc.at[pl.ds(sid * RPT, RPT)],
                    out_hbm.at[pl.ds(cid * NP + sid * RPT, RPT)])


@functools.cache
def _agg():
    return functools.partial(
        pl.kernel,
        mesh=plsc.VectorSubcoreMesh(core_axis_name="c", subcore_axis_name="s"),
        out_type=jax.ShapeDtypeStruct((NC * NP, D), jnp.float32),
        scratch_types=[
            pltpu.VMEM((QCH, CH), jnp.int32),
            pltpu.VMEM((QCH, CH), jnp.int32),
            pltpu.VMEM((CH, D), jnp.float32),
            pltpu.VMEM((CH, D), jnp.float32),
            pltpu.VMEM_SHARED((NP, D), jnp.float32),
            pltpu.SemaphoreType.DMA,
            pltpu.SemaphoreType.DMA,
        ],
    )(_agg_body)


def _pool_body(h_hbm, bidx_hbm, out_hbm, bidxv, hrows, onesr, pacc, cacc):
    cid = lax.axis_index("c")
    sid = lax.axis_index("s")
    wid = sid * NC + cid
    spt = SEG // NS

    def fill_zero(i, carry):
        hrows[i // 8, pl.ds((i % 8) * 16, 16)] = _zero16()
        return carry
    lax.fori_loop(0, CHP * 8, fill_zero, 0)
    pltpu.sync_copy(hrows.at[pl.ds(0, spt)], pacc.at[pl.ds(sid * spt, spt)])
    pltpu.sync_copy(hrows.at[pl.ds(0, spt)], cacc.at[pl.ds(sid * spt, spt)])

    def fill_ones(i, carry):
        onesr[i // 8, pl.ds((i % 8) * 16, 16)] = jnp.full((16,), 1.0, jnp.float32)
        return carry
    lax.fori_loop(0, CHP * 8, fill_ones, 0)

    pltpu.sync_copy(bidx_hbm.at[pl.ds(wid * NCHP, NCHP)], bidxv)
    plsc.subcore_barrier()

    def body(g, carry):
        pltpu.sync_copy(
            h_hbm.at[pl.ds(wid * (NP // NW) + g * CHP, CHP)], hrows)
        pltpu.sync_copy(hrows, pacc.at[bidxv.at[g]], add=True)
        pltpu.sync_copy(onesr, cacc.at[bidxv.at[g]], add=True)
        return carry
    lax.fori_loop(0, NCHP, body, 0)

    plsc.subcore_barrier()
    pltpu.sync_copy(pacc.at[pl.ds(sid * spt, spt)],
                    out_hbm.at[pl.ds(cid * 2 * SEG + sid * spt, spt)])
    pltpu.sync_copy(cacc.at[pl.ds(sid * spt, spt)],
                    out_hbm.at[pl.ds(cid * 2 * SEG + SEG + sid * spt, spt)])


@functools.cache
def _pool():
    return functools.partial(
        pl.kernel,
        mesh=plsc.VectorSubcoreMesh(core_axis_name="c", subcore_axis_name="s"),
        out_type=jax.ShapeDtypeStruct((NC * 2 * SEG, D), jnp.float32),
        scratch_types=[
            pltpu.VMEM((NCHP, CHP), jnp.int32),
            pltpu.VMEM((CHP, D), jnp.float32),
            pltpu.VMEM((CHP, D), jnp.float32),
            pltpu.VMEM_SHARED((SEG, D), jnp.float32),
            pltpu.VMEM_SHARED((SEG, D), jnp.float32),
        ],
    )(_pool_body)


def _mm_scaled_body(x_ref, w_ref, deg_ref, y_ref):
    deg = jnp.sum(deg_ref[...], axis=0) + 1.0
    dinv = lax.rsqrt(deg)
    y_ref[...] = jnp.dot(x_ref[...], w_ref[...],
                         preferred_element_type=jnp.float32) * dinv[:, None]


def _mm_scaled(x, w, deg2):
    return pl.pallas_call(
        _mm_scaled_body,
        grid=(NB,),
        in_specs=[
            pl.BlockSpec((BM, D), lambda i: (i, 0)),
            pl.BlockSpec((D, H), lambda i: (0, 0)),
            pl.BlockSpec((NC, BM), lambda i: (0, i)),
        ],
        out_specs=pl.BlockSpec((BM, H), lambda i: (i, 0)),
        out_shape=jax.ShapeDtypeStruct((NP, H), jnp.float32),
    )(x, w, deg2)


def _stats_body(accs_ref, y_ref, deg_ref, b_ref, z_ref, ps_ref, pq_ref):
    i = pl.program_id(0)
    deg = jnp.sum(deg_ref[...], axis=0) + 1.0
    dinv = lax.rsqrt(deg)
    a = accs_ref[0] + accs_ref[1] + y_ref[...]
    z = a * dinv[:, None] + b_ref[...]
    z_ref[...] = z
    ridx = lax.broadcasted_iota(jnp.int32, (BM, 1), 0) + i * BM
    zm = jnp.where(ridx < N, z, 0.0)
    ps_ref[...] = jnp.sum(zm, axis=0).reshape(1, 1, H)
    pq_ref[...] = jnp.sum(zm * zm, axis=0).reshape(1, 1, H)


def _stats(accs, y, deg2, b):
    return pl.pallas_call(
        _stats_body,
        grid=(NB,),
        in_specs=[
            pl.BlockSpec((NC, BM, H), lambda i: (0, i, 0)),
            pl.BlockSpec((BM, H), lambda i: (i, 0)),
            pl.BlockSpec((NC, BM), lambda i: (0, i)),
            pl.BlockSpec((1, H), lambda i: (0, 0)),
        ],
        out_specs=[
            pl.BlockSpec((BM, H), lambda i: (i, 0)),
            pl.BlockSpec((1, 1, H), lambda i: (i, 0, 0)),
            pl.BlockSpec((1, 1, H), lambda i: (i, 0, 0)),
        ],
        out_shape=[
            jax.ShapeDtypeStruct((NP, H), jnp.float32),
            jax.ShapeDtypeStruct((NB, 1, H), jnp.float32),
            jax.ShapeDtypeStruct((NB, 1, H), jnp.float32),
        ],
    )(accs, y, deg2, b)


def _bn_common(z_ref, ps_ref, pq_ref, g_ref, bt_ref):
    s = jnp.sum(ps_ref[...], axis=(0, 1))
    q = jnp.sum(pq_ref[...], axis=(0, 1))
    mean = s * (1.0 / N)
    var = q * (1.0 / N) - mean * mean
    scale = lax.rsqrt(var + EPS) * g_ref[0]
    return jnp.maximum((z_ref[...] - mean) * scale + bt_ref[0], 0.0)


def _norm_mm_body(z_ref, ps_ref, pq_ref, g_ref, bt_ref, w_ref, deg_ref, y_ref):
    h = _bn_common(z_ref, ps_ref, pq_ref, g_ref, bt_ref)
    deg = jnp.sum(deg_ref[...], axis=0) + 1.0
    dinv = lax.rsqrt(deg)
    y_ref[...] = jnp.dot(h, w_ref[...],
                         preferred_element_type=jnp.float32) * dinv[:, None]


def _norm_mm(z, ps, pq, g, bt, w, deg2):
    return pl.pallas_call(
        _norm_mm_body,
        grid=(NB,),
        in_specs=[
            pl.BlockSpec((BM, H), lambda i: (i, 0)),
            pl.BlockSpec((NB, 1, H), lambda i: (0, 0, 0)),
            pl.BlockSpec((NB, 1, H), lambda i: (0, 0, 0)),
            pl.BlockSpec((1, H), lambda i: (0, 0)),
            pl.BlockSpec((1, H), lambda i: (0, 0)),
            pl.BlockSpec((H, H), lambda i: (0, 0)),
            pl.BlockSpec((NC, BM), lambda i: (0, i)),
        ],
        out_specs=pl.BlockSpec((BM, H), lambda i: (i, 0)),
        out_shape=jax.ShapeDtypeStruct((NP, H), jnp.float32),
    )(z, ps, pq, g, bt, w, deg2)


def _norm_only_body(z_ref, ps_ref, pq_ref, g_ref, bt_ref, h_ref):
    h_ref[...] = _bn_common(z_ref, ps_ref, pq_ref, g_ref, bt_ref)


def _norm_only(z, ps, pq, g, bt):
    return pl.pallas_call(
        _norm_only_body,
        grid=(NB,),
        in_specs=[
            pl.BlockSpec((BM, H), lambda i: (i, 0)),
            pl.BlockSpec((NB, 1, H), lambda i: (0, 0, 0)),
            pl.BlockSpec((NB, 1, H), lambda i: (0, 0, 0)),
            pl.BlockSpec((1, H), lambda i: (0, 0)),
            pl.BlockSpec((1, H), lambda i: (0, 0)),
        ],
        out_specs=pl.BlockSpec((BM, H), lambda i: (i, 0)),
        out_shape=jax.ShapeDtypeStruct((NP, H), jnp.float32),
    )(z, ps, pq, g, bt)


def _fc_body(p0_ref, c0_ref, p1_ref, c1_ref, w_ref, b_ref, o_ref):
    psum = p0_ref[...] + p1_ref[...]
    cnt = jnp.maximum(c0_ref[...] + c1_ref[...], 1.0)
    pool = psum / cnt
    o_ref[...] = jnp.maximum(
        jnp.dot(pool[:B], w_ref[...], preferred_element_type=jnp.float32)
        + b_ref[...], 0.0)


def _fc(p0, c0, p1, c1, w, b):
    return pl.pallas_call(
        _fc_body,
        grid=(1,),
        in_specs=[
            pl.BlockSpec((SEG, H), lambda i: (0, 0)),
            pl.BlockSpec((SEG, H), lambda i: (0, 0)),
            pl.BlockSpec((SEG, H), lambda i: (0, 0)),
            pl.BlockSpec((SEG, H), lambda i: (0, 0)),
            pl.BlockSpec((H, OUT), lambda i: (0, 0)),
            pl.BlockSpec((1, OUT), lambda i: (0, 0)),
        ],
        out_specs=pl.BlockSpec((B, OUT), lambda i: (0, 0)),
        out_shape=jax.ShapeDtypeStruct((B, OUT), jnp.float32),
    )(p0, c0, p1, c1, w, b)


def kernel(x_all, edge_index_all, batch_index, W1, b1, g1, bt1, W2, b2, g2,
           bt2, W3, b3, g3, bt3, Wfc, bfc):
    epad = jnp.full((2, EPAD - E), N, jnp.int32)
    ei = jnp.concatenate([edge_index_all, epad], axis=1)
    src2 = ei[0].reshape(EPAD // CH, CH)
    dst2 = ei[1].reshape(EPAD // CH, CH)
    x_pad = jnp.concatenate(
        [x_all.astype(jnp.float32), jnp.zeros((NP - N, D), jnp.float32)], 0)
    bidx2 = jnp.concatenate(
        [batch_index, jnp.full((NP - N,), B, jnp.int32)], 0).reshape(
            NP // CHP, CHP)

    deg2 = _deg()(dst2).reshape(NC, NP)

    b1r, g1r, bt1r = b1.reshape(1, H), g1.reshape(1, H), bt1.reshape(1, H)
    b2r, g2r, bt2r = b2.reshape(1, H), g2.reshape(1, H), bt2.reshape(1, H)
    b3r, g3r, bt3r = b3.reshape(1, H), g3.reshape(1, H), bt3.reshape(1, H)

    y1 = _mm_scaled(x_pad, W1, deg2)
    acc1 = _agg()(y1, src2, dst2).reshape(NC, NP, D)
    z1, ps1, pq1 = _stats(acc1, y1, deg2, b1r)
    y2 = _norm_mm(z1, ps1, pq1, g1r, bt1r, W2, deg2)

    acc2 = _agg()(y2, src2, dst2).reshape(NC, NP, D)
    z2, ps2, pq2 = _stats(acc2, y2, deg2, b2r)
    y3 = _norm_mm(z2, ps2, pq2, g2r, bt2r, W3, deg2)

    acc3 = _agg()(y3, src2, dst2).reshape(NC, NP, D)
    z3, ps3, pq3 = _stats(acc3, y3, deg2, b3r)
    h3 = _norm_only(z3, ps3, pq3, g3r, bt3r)

    pool_flat = _pool()(h3, bidx2)
    p0 = pool_flat[0:SEG]
    c0 = pool_flat[SEG:2 * SEG]
    p1 = pool_flat[2 * SEG:3 * SEG]
    c1 = pool_flat[3 * SEG:4 * SEG]

    return _fc(p0, c0, p1, c1, Wfc, bfc.reshape(1, OUT))
